# baseline (speedup 1.0000x reference)
import functools

import jax
import jax.numpy as jnp
from jax import lax
from jax.experimental import pallas as pl
from jax.experimental.pallas import tpu as pltpu
from jax.experimental.pallas import tpu_sc as plsc

N = 10000
E = 320000
D = 128
DH = 64
AB = 16
KT5 = 900

NC = 2
NS = 16
NW = NC * NS
EPW = E // NW
B = 80
CH = EPW // B
RSTEP = 624
RPT = 640
BR = 1000
EB = E // (N // BR)


def _pre_body(x_ref, wmsg_ref, wself_ref, wr2_ref, y_ref, h0_ref, ab_ref):
    xb = x_ref[...]
    y_ref[...] = jnp.dot(xb, wmsg_ref[...], preferred_element_type=jnp.float32)
    h0_ref[...] = jnp.dot(xb, wself_ref[...], preferred_element_type=jnp.float32)
    ab_ref[...] = jnp.dot(xb, wr2_ref[...], preferred_element_type=jnp.float32)


def _pack_body(ei_ref, pck_ref):
    pck_ref[...] = ei_ref[0] | (ei_ref[1] << 16)


def _tail_body(h0_ref, p0_ref, p1_ref, wupd_ref, wdec_ref, out_ref):
    agg = p0_ref[0] + p1_ref[0]
    h = (h0_ref[...]
         + jnp.dot(agg, wupd_ref[...], preferred_element_type=jnp.float32))
    h = jnp.maximum(h, 0.0)
    out_ref[...] = jnp.dot(h, wdec_ref[...], preferred_element_type=jnp.float32)


def _sc_body(pck_hbm, y_hbm, ab_hbm, out_hbm,
             agg_sh, pck_v, src0, dst0, src1, dst1, abs0, abd0, rows0, srows0,
             abs1, abd1, rows1, srows1, gates_v,
             sem0, sem1, sems0, sems1, semi):
    cid = lax.axis_index("c")
    sid = lax.axis_index("s")
    wid = cid * NS + sid
    base_n = sid * RSTEP

    def _pre_issue(t, c):
        pltpu.async_copy(pck_hbm.at[pl.ds(wid * EPW + t * B, B)],
                         pck_v.at[t], semi)
        return c
    lax.fori_loop(0, CH, _pre_issue, 0)

    def _pre_drain(t, c):
        pltpu.make_async_copy(pck_hbm.at[pl.ds(wid * EPW + t * B, B)],
                              pck_v.at[t], semi).wait()
        return c
    lax.fori_loop(0, CH, _pre_drain, 0)

    src_b, dst_b = (src0, src1), (dst0, dst1)
    abs_b, abd_b = (abs0, abs1), (abd0, abd1)
    rows_b, srows_b = (rows0, rows1), (srows0, srows1)
    sem_b, sems_b = (sem0, sem1), (sems0, sems1)

    for half in range(2):
        def _zrow(r, c):
            for j in range(DH // 16):
                srows0[r, pl.ds(j * 16, 16)] = jnp.zeros((16,), jnp.float32)
            return c
        lax.fori_loop(0, B, _zrow, 0)
        for i in range(RPT // B):
            pltpu.sync_copy(srows0, agg_sh.at[pl.ds(base_n + i * B, B)])
        plsc.subcore_barrier()

        def issue(t, b):
            @pl.when(t >= 2)
            def _():
                pltpu.make_async_copy(srows_b[b], agg_sh.at[dst_b[b]],
                                      sems_b[b]).wait()
            for g in range(B // 16):
                p16 = pck_v[t, pl.ds(g * 16, 16)]
                src_b[b][pl.ds(g * 16, 16)] = p16 & 0xFFFF
                dst_b[b][pl.ds(g * 16, 16)] = lax.shift_right_logical(p16, 16)
            pltpu.async_copy(ab_hbm.at[src_b[b]], abs_b[b], sem_b[b])
            pltpu.async_copy(ab_hbm.at[dst_b[b]], abd_b[b], sem_b[b])
            pltpu.async_copy(y_hbm.at[src_b[b]], rows_b[b], sem_b[b])

        def process(t, b):
            pltpu.make_async_copy(ab_hbm.at[src_b[b]], abs_b[b],
                                  sem_b[b]).wait()
            pltpu.make_async_copy(ab_hbm.at[dst_b[b]], abd_b[b],
                                  sem_b[b]).wait()
            pltpu.make_async_copy(y_hbm.at[src_b[b]], rows_b[b],
                                  sem_b[b]).wait()
            for g in range(B // 16):
                ev = jnp.arange(16, dtype=jnp.int32) + (g * 16)
                l = []
                for k in range(4):
                    a_k = plsc.load_gather(
                        abs_b[b], [ev, jnp.full((16,), k, jnp.int32)])
                    b_k = plsc.load_gather(
                        abd_b[b], [ev, jnp.full((16,), 4 + k, jnp.int32)])
                    l.append(a_k + b_k)
                m = jnp.maximum(jnp.maximum(l[0], l[1]),
                                jnp.maximum(l[2], l[3]))
                e0 = jnp.exp(l[0] - m)
                s123 = (jnp.exp(l[1] - m) + jnp.exp(l[2] - m)
                        + jnp.exp(l[3] - m))
                gates_v[pl.ds(g * 16, 16)] = s123 / (e0 + s123)

            rv, sv = rows_b[b], srows_b[b]
            col = half * DH

            @plsc.parallel_loop(0, B, 1, unroll=8)
            def scale_row(e2):
                gsc = plsc.load_gather(
                    gates_v, [jnp.full((16,), 0, jnp.int32) + e2])
                for j in range(DH // 16):
                    sv[e2, pl.ds(j * 16, 16)] = (
                        rv[e2, pl.ds(col + j * 16, 16)] * gsc)

            pltpu.make_async_copy(srows_b[b], agg_sh.at[dst_b[b]],
                                  sems_b[b]).start(add=True)

        issue(0, 0)

        def pair(p, carry):
            t1 = 2 * p + 1
            issue(t1, 1)
            process(2 * p, 0)
            issue(t1 + 1, 0)
            process(t1, 1)
            return carry

        lax.fori_loop(0, (CH - 1) // 2, pair, 0)
        process(CH - 1, 0)
        pltpu.make_async_copy(srows_b[0], agg_sh.at[dst_b[0]],
                              sems_b[0]).wait()
        pltpu.make_async_copy(srows_b[1], agg_sh.at[dst_b[1]],
                              sems_b[1]).wait()
        plsc.subcore_barrier()
        pltpu.sync_copy(agg_sh.at[pl.ds(base_n, RPT)],
                        out_hbm.at[cid, pl.ds(base_n, RPT),
                                   pl.ds(half * DH, DH)])
        plsc.subcore_barrier()


@functools.cache
def _sc_agg():
    return pl.kernel(
        _sc_body,
        out_type=jax.ShapeDtypeStruct((NC, N, D), jnp.float32),
        mesh=plsc.VectorSubcoreMesh(core_axis_name="c", subcore_axis_name="s",
                                    num_cores=NC, num_subcores=NS),
        scratch_types=[
            pltpu.VMEM_SHARED((N, DH), jnp.float32),
            pltpu.VMEM((CH, B), jnp.int32),
            pltpu.VMEM((B,), jnp.int32),
            pltpu.VMEM((B,), jnp.int32),
            pltpu.VMEM((B,), jnp.int32),
            pltpu.VMEM((B,), jnp.int32),
            pltpu.VMEM((B, AB), jnp.float32),
            pltpu.VMEM((B, AB), jnp.float32),
            pltpu.VMEM((B, D), jnp.float32),
            pltpu.VMEM((B, DH), jnp.float32),
            pltpu.VMEM((B, AB), jnp.float32),
            pltpu.VMEM((B, AB), jnp.float32),
            pltpu.VMEM((B, D), jnp.float32),
            pltpu.VMEM((B, DH), jnp.float32),
            pltpu.VMEM((B,), jnp.float32),
            pltpu.SemaphoreType.DMA,
            pltpu.SemaphoreType.DMA,
            pltpu.SemaphoreType.DMA,
            pltpu.SemaphoreType.DMA,
            pltpu.SemaphoreType.DMA,
        ],
        compiler_params=pltpu.CompilerParams(needs_layout_passes=False,
                                             use_tc_tiling_on_sc=False),
    )


@jax.jit
def kernel(x, edge_index, W_rel, W_msg, W_self, W_upd, W_dec):
    wr2 = jnp.concatenate(
        [W_rel[:D], W_rel[D:], jnp.zeros((D, AB - 8), jnp.float32)], axis=1)

    y, h0, ab = pl.pallas_call(
        _pre_body,
        grid=(N // BR,),
        in_specs=[
            pl.BlockSpec((BR, D), lambda i: (i, 0)),
            pl.BlockSpec((D, D), lambda i: (0, 0)),
            pl.BlockSpec((D, D), lambda i: (0, 0)),
            pl.BlockSpec((D, AB), lambda i: (0, 0)),
        ],
        out_specs=[
            pl.BlockSpec((BR, D), lambda i: (i, 0)),
            pl.BlockSpec((BR, D), lambda i: (i, 0)),
            pl.BlockSpec((BR, AB), lambda i: (i, 0)),
        ],
        out_shape=[
            jax.ShapeDtypeStruct((N, D), jnp.float32),
            jax.ShapeDtypeStruct((N, D), jnp.float32),
            jax.ShapeDtypeStruct((N, AB), jnp.float32),
        ],
    )(x, W_msg, W_self, wr2)

    pck = pl.pallas_call(
        _pack_body,
        out_shape=jax.ShapeDtypeStruct((E,), jnp.int32),
    )(edge_index)

    parts = _sc_agg()(pck, y, ab)

    out = pl.pallas_call(
        _tail_body,
        grid=(N // BR,),
        in_specs=[
            pl.BlockSpec((BR, D), lambda i: (i, 0)),
            pl.BlockSpec((1, BR, D), lambda i: (0, i, 0)),
            pl.BlockSpec((1, BR, D), lambda i: (1, i, 0)),
            pl.BlockSpec((D, D), lambda i: (0, 0)),
            pl.BlockSpec((D, KT5), lambda i: (0, 0)),
        ],
        out_specs=pl.BlockSpec((BR, KT5), lambda i: (i, 0)),
        out_shape=jax.ShapeDtypeStruct((N, KT5), jnp.float32),
    )(h0, parts, parts, W_upd, W_dec)

    return out.reshape(N, 6, 30, 5)


# speedup vs baseline: 1.3405x; 1.3405x over previous
import functools

import jax
import jax.numpy as jnp
from jax import lax
from jax.experimental import pallas as pl
from jax.experimental.pallas import tpu as pltpu
from jax.experimental.pallas import tpu_sc as plsc

N = 10000
E = 320000
D = 128
DH = 64
AB = 16
K = 6
T = 30
F = 5

NC = 2
NS = 16
NW = NC * NS
EPW = E // NW
B = 80
CH = EPW // B
RSTEP = 624
RPT = 640
BR = 1000


def _pre_body(x_ref, wmsg_ref, wself_ref, wr2_ref, y0_ref, y1_ref, h0_ref,
              ab_ref):
    xb = x_ref[...]
    ym = jnp.dot(xb, wmsg_ref[...], preferred_element_type=jnp.float32)
    y0_ref[...] = ym[:, :DH]
    y1_ref[...] = ym[:, DH:]
    h0_ref[...] = jnp.dot(xb, wself_ref[...], preferred_element_type=jnp.float32)
    ab_ref[...] = jnp.dot(xb, wr2_ref[...], preferred_element_type=jnp.float32)


def _pack_body(ei_ref, pck_ref):
    pck_ref[...] = ei_ref[0] | (ei_ref[1] << 16)


def _tail_body(h0_ref, p00_ref, p01_ref, p10_ref, p11_ref, wu0_ref, wu1_ref,
               h_ref):
    agg0 = p00_ref[0, 0] + p10_ref[0, 0]
    agg1 = p01_ref[0, 0] + p11_ref[0, 0]
    h = (h0_ref[...]
         + jnp.dot(agg0, wu0_ref[...], preferred_element_type=jnp.float32)
         + jnp.dot(agg1, wu1_ref[...], preferred_element_type=jnp.float32))
    h_ref[...] = jnp.maximum(h, 0.0)


def _dec_body(h_ref, wp_ref, out_ref):
    for f in range(F):
        out_ref[0, f] = lax.dot_general(
            wp_ref[0, f], h_ref[...], (((0,), (1,)), ((), ())),
            preferred_element_type=jnp.float32)


def _sc_body(pck_hbm, y0_hbm, y1_hbm, ab_hbm, out_hbm,
             agg_sh, pck_v, src0, dst0, src1, dst1, abs0, abd0, rows0,
             abs1, abd1, rows1, gates_v,
             sem0, sem1, sems0, sems1, semi):
    cid = lax.axis_index("c")
    sid = lax.axis_index("s")
    wid = cid * NS + sid
    base_n = sid * RSTEP

    def _pre_issue(t, c):
        pltpu.async_copy(pck_hbm.at[pl.ds(wid * EPW + t * B, B)],
                         pck_v.at[t], semi)
        return c
    lax.fori_loop(0, CH, _pre_issue, 0)

    def _pre_drain(t, c):
        pltpu.make_async_copy(pck_hbm.at[pl.ds(wid * EPW + t * B, B)],
                              pck_v.at[t], semi).wait()
        return c
    lax.fori_loop(0, CH, _pre_drain, 0)

    src_b, dst_b = (src0, src1), (dst0, dst1)
    abs_b, abd_b = (abs0, abs1), (abd0, abd1)
    rows_b = (rows0, rows1)
    sem_b, sems_b = (sem0, sem1), (sems0, sems1)

    for half in range(2):
        yh_hbm = y0_hbm if half == 0 else y1_hbm
        def _zrow(r, c):
            for j in range(DH // 16):
                rows0[r, pl.ds(j * 16, 16)] = jnp.zeros((16,), jnp.float32)
            return c
        lax.fori_loop(0, B, _zrow, 0)
        for i in range(RPT // B):
            pltpu.sync_copy(rows0, agg_sh.at[pl.ds(base_n + i * B, B)])
        plsc.subcore_barrier()

        def issue(t, b):
            @pl.when(t >= 2)
            def _():
                pltpu.make_async_copy(rows_b[b], agg_sh.at[dst_b[b]],
                                      sems_b[b]).wait()
            for g in range(B // 16):
                p16 = pck_v[t, pl.ds(g * 16, 16)]
                src_b[b][pl.ds(g * 16, 16)] = p16 & 0xFFFF
                dst_b[b][pl.ds(g * 16, 16)] = lax.shift_right_logical(p16, 16)
            pltpu.async_copy(ab_hbm.at[src_b[b]], abs_b[b], sem_b[b])
            pltpu.async_copy(ab_hbm.at[dst_b[b]], abd_b[b], sem_b[b])
            pltpu.async_copy(yh_hbm.at[src_b[b]], rows_b[b], sem_b[b])

        def process(t, b):
            pltpu.make_async_copy(ab_hbm.at[src_b[b]], abs_b[b],
                                  sem_b[b]).wait()
            pltpu.make_async_copy(ab_hbm.at[dst_b[b]], abd_b[b],
                                  sem_b[b]).wait()
            pltpu.make_async_copy(yh_hbm.at[src_b[b]], rows_b[b],
                                  sem_b[b]).wait()
            for g in range(B // 16):
                ev = jnp.arange(16, dtype=jnp.int32) + (g * 16)
                l = []
                for k in range(4):
                    a_k = plsc.load_gather(
                        abs_b[b], [ev, jnp.full((16,), k, jnp.int32)])
                    b_k = plsc.load_gather(
                        abd_b[b], [ev, jnp.full((16,), 4 + k, jnp.int32)])
                    l.append(a_k + b_k)
                m = jnp.maximum(jnp.maximum(l[0], l[1]),
                                jnp.maximum(l[2], l[3]))
                e0 = jnp.exp(l[0] - m)
                s123 = (jnp.exp(l[1] - m) + jnp.exp(l[2] - m)
                        + jnp.exp(l[3] - m))
                gates_v[pl.ds(g * 16, 16)] = s123 / (e0 + s123)

            rv = rows_b[b]

            @plsc.parallel_loop(0, B, 1, unroll=8)
            def scale_row(e2):
                gsc = plsc.load_gather(
                    gates_v, [jnp.full((16,), 0, jnp.int32) + e2])
                for j in range(DH // 16):
                    rv[e2, pl.ds(j * 16, 16)] = rv[e2, pl.ds(j * 16, 16)] * gsc

            pltpu.make_async_copy(rows_b[b], agg_sh.at[dst_b[b]],
                                  sems_b[b]).start(add=True)

        issue(0, 0)

        def pair(p, carry):
            t1 = 2 * p + 1
            issue(t1, 1)
            process(2 * p, 0)
            issue(t1 + 1, 0)
            process(t1, 1)
            return carry

        lax.fori_loop(0, (CH - 1) // 2, pair, 0)
        process(CH - 1, 0)
        pltpu.make_async_copy(rows_b[0], agg_sh.at[dst_b[0]],
                              sems_b[0]).wait()
        pltpu.make_async_copy(rows_b[1], agg_sh.at[dst_b[1]],
                              sems_b[1]).wait()
        plsc.subcore_barrier()
        pltpu.sync_copy(agg_sh.at[pl.ds(base_n, RPT)],
                        out_hbm.at[cid, half, pl.ds(base_n, RPT)])
        plsc.subcore_barrier()


@functools.cache
def _sc_agg():
    return pl.kernel(
        _sc_body,
        out_type=jax.ShapeDtypeStruct((NC, 2, N, DH), jnp.float32),
        mesh=plsc.VectorSubcoreMesh(core_axis_name="c", subcore_axis_name="s",
                                    num_cores=NC, num_subcores=NS),
        scratch_types=[
            pltpu.VMEM_SHARED((N, DH), jnp.float32),
            pltpu.VMEM((CH, B), jnp.int32),
            pltpu.VMEM((B,), jnp.int32),
            pltpu.VMEM((B,), jnp.int32),
            pltpu.VMEM((B,), jnp.int32),
            pltpu.VMEM((B,), jnp.int32),
            pltpu.VMEM((B, AB), jnp.float32),
            pltpu.VMEM((B, AB), jnp.float32),
            pltpu.VMEM((B, DH), jnp.float32),
            pltpu.VMEM((B, AB), jnp.float32),
            pltpu.VMEM((B, AB), jnp.float32),
            pltpu.VMEM((B, DH), jnp.float32),
            pltpu.VMEM((B,), jnp.float32),
            pltpu.SemaphoreType.DMA,
            pltpu.SemaphoreType.DMA,
            pltpu.SemaphoreType.DMA,
            pltpu.SemaphoreType.DMA,
            pltpu.SemaphoreType.DMA,
        ],
        compiler_params=pltpu.CompilerParams(needs_layout_passes=False,
                                             use_tc_tiling_on_sc=False),
    )


@jax.jit
def kernel(x, edge_index, W_rel, W_msg, W_self, W_upd, W_dec):
    wr2 = jnp.concatenate(
        [W_rel[:D], W_rel[D:], jnp.zeros((D, AB - 8), jnp.float32)], axis=1)
    wp = jnp.transpose(W_dec.reshape(D, K, T, F), (1, 3, 0, 2))

    y0, y1, h0, ab = pl.pallas_call(
        _pre_body,
        grid=(N // BR,),
        in_specs=[
            pl.BlockSpec((BR, D), lambda i: (i, 0)),
            pl.BlockSpec((D, D), lambda i: (0, 0)),
            pl.BlockSpec((D, D), lambda i: (0, 0)),
            pl.BlockSpec((D, AB), lambda i: (0, 0)),
        ],
        out_specs=[
            pl.BlockSpec((BR, DH), lambda i: (i, 0)),
            pl.BlockSpec((BR, DH), lambda i: (i, 0)),
            pl.BlockSpec((BR, D), lambda i: (i, 0)),
            pl.BlockSpec((BR, AB), lambda i: (i, 0)),
        ],
        out_shape=[
            jax.ShapeDtypeStruct((N, DH), jnp.float32),
            jax.ShapeDtypeStruct((N, DH), jnp.float32),
            jax.ShapeDtypeStruct((N, D), jnp.float32),
            jax.ShapeDtypeStruct((N, AB), jnp.float32),
        ],
    )(x, W_msg, W_self, wr2)

    pck = pl.pallas_call(
        _pack_body,
        out_shape=jax.ShapeDtypeStruct((E,), jnp.int32),
    )(edge_index)

    parts = _sc_agg()(pck, y0, y1, ab)

    h = pl.pallas_call(
        _tail_body,
        grid=(N // BR,),
        in_specs=[
            pl.BlockSpec((BR, D), lambda i: (i, 0)),
            pl.BlockSpec((1, 1, BR, DH), lambda i: (0, 0, i, 0)),
            pl.BlockSpec((1, 1, BR, DH), lambda i: (0, 1, i, 0)),
            pl.BlockSpec((1, 1, BR, DH), lambda i: (1, 0, i, 0)),
            pl.BlockSpec((1, 1, BR, DH), lambda i: (1, 1, i, 0)),
            pl.BlockSpec((DH, D), lambda i: (0, 0)),
            pl.BlockSpec((DH, D), lambda i: (1, 0)),
        ],
        out_specs=pl.BlockSpec((BR, D), lambda i: (i, 0)),
        out_shape=jax.ShapeDtypeStruct((N, D), jnp.float32),
    )(h0, parts, parts, parts, parts, W_upd, W_upd)

    out4 = pl.pallas_call(
        _dec_body,
        grid=(K,),
        in_specs=[
            pl.BlockSpec((N, D), lambda k: (0, 0)),
            pl.BlockSpec((1, F, D, T), lambda k: (k, 0, 0, 0)),
        ],
        out_specs=pl.BlockSpec((1, F, T, N), lambda k: (k, 0, 0, 0)),
        out_shape=jax.ShapeDtypeStruct((K, F, T, N), jnp.float32),
    )(h, wp)

    return jnp.transpose(out4, (3, 0, 2, 1))


# speedup vs baseline: 1.4692x; 1.0960x over previous
import functools

import jax
import jax.numpy as jnp
from jax import lax
from jax.experimental import pallas as pl
from jax.experimental.pallas import tpu as pltpu
from jax.experimental.pallas import tpu_sc as plsc

N = 10000
E = 320000
D = 128
DH = 64
AB = 16
K = 6
T = 30
F = 5

NC = 2
NS = 16
NW = NC * NS
EPW = E // NW
B = 80
CH = EPW // B
RSTEP = 624
RPT = 640
BR = 1000


def _pre_body(x_ref, wmsg_ref, wself_ref, wr2_ref, y0_ref, y1_ref, h0_ref,
              ab_ref):
    xb = x_ref[...]
    ym = jnp.dot(xb, wmsg_ref[...], preferred_element_type=jnp.float32)
    y0_ref[...] = ym[:, :DH]
    y1_ref[...] = ym[:, DH:]
    h0_ref[...] = jnp.dot(xb, wself_ref[...], preferred_element_type=jnp.float32)
    ab_ref[...] = jnp.exp(
        jnp.dot(xb, wr2_ref[...], preferred_element_type=jnp.float32))


def _tail_body(h0_ref, p00_ref, p01_ref, p10_ref, p11_ref, wu0_ref, wu1_ref,
               h_ref):
    agg0 = p00_ref[0, 0] + p10_ref[0, 0]
    agg1 = p01_ref[0, 0] + p11_ref[0, 0]
    h = (h0_ref[...]
         + jnp.dot(agg0, wu0_ref[...], preferred_element_type=jnp.float32)
         + jnp.dot(agg1, wu1_ref[...], preferred_element_type=jnp.float32))
    h_ref[...] = jnp.maximum(h, 0.0)


def _dec_body(h_ref, wp_ref, out_ref):
    for f in range(F):
        out_ref[0, f] = lax.dot_general(
            wp_ref[0, f], h_ref[...], (((0,), (1,)), ((), ())),
            preferred_element_type=jnp.float32)


def _sc_body(ei_hbm, y0_hbm, y1_hbm, ab_hbm, out_hbm,
             agg_sh, srcs_v, dsts_v, gates_v, abs0, abd0, rows0,
             abs1, abd1, rows1,
             sem0, sem1, sems0, sems1, semi):
    cid = lax.axis_index("c")
    sid = lax.axis_index("s")
    wid = cid * NS + sid
    base_n = sid * RSTEP

    def _pre_issue(t, c):
        off = wid * EPW + t * B
        pltpu.async_copy(ei_hbm.at[0, pl.ds(off, B)], srcs_v.at[t], semi)
        pltpu.async_copy(ei_hbm.at[1, pl.ds(off, B)], dsts_v.at[t], semi)
        return c
    lax.fori_loop(0, CH, _pre_issue, 0)

    def _pre_drain(t, c):
        off = wid * EPW + t * B
        pltpu.make_async_copy(ei_hbm.at[0, pl.ds(off, B)], srcs_v.at[t],
                              semi).wait()
        pltpu.make_async_copy(ei_hbm.at[1, pl.ds(off, B)], dsts_v.at[t],
                              semi).wait()
        return c
    lax.fori_loop(0, CH, _pre_drain, 0)

    abs_b, abd_b = (abs0, abs1), (abd0, abd1)
    rows_b = (rows0, rows1)
    sem_b, sems_b = (sem0, sem1), (sems0, sems1)

    for half in range(2):
        yh_hbm = y0_hbm if half == 0 else y1_hbm
        def _zrow(r, c):
            for j in range(DH // 16):
                rows0[r, pl.ds(j * 16, 16)] = jnp.zeros((16,), jnp.float32)
            return c
        lax.fori_loop(0, B, _zrow, 0)
        for i in range(RPT // B):
            pltpu.sync_copy(rows0, agg_sh.at[pl.ds(base_n + i * B, B)])
        plsc.subcore_barrier()

        def issue(t, b):
            @pl.when(t >= 2)
            def _():
                pltpu.make_async_copy(rows_b[b], agg_sh.at[dsts_v.at[t]],
                                      sems_b[b]).wait()
            if half == 0:
                pltpu.async_copy(ab_hbm.at[srcs_v.at[t]], abs_b[b], sem_b[b])
                pltpu.async_copy(ab_hbm.at[dsts_v.at[t]], abd_b[b], sem_b[b])
            pltpu.async_copy(yh_hbm.at[srcs_v.at[t]], rows_b[b], sem_b[b])

        def process(t, b):
            if half == 0:
                pltpu.make_async_copy(ab_hbm.at[srcs_v.at[t]], abs_b[b],
                                      sem_b[b]).wait()
                pltpu.make_async_copy(ab_hbm.at[dsts_v.at[t]], abd_b[b],
                                      sem_b[b]).wait()
            pltpu.make_async_copy(yh_hbm.at[srcs_v.at[t]], rows_b[b],
                                  sem_b[b]).wait()
            if half == 0:
                for g in range(B // 16):
                    ev = jnp.arange(16, dtype=jnp.int32) + (g * 16)
                    p = []
                    for k in range(4):
                        ea_k = plsc.load_gather(
                            abs_b[b], [ev, jnp.full((16,), k, jnp.int32)])
                        eb_k = plsc.load_gather(
                            abd_b[b], [ev, jnp.full((16,), 4 + k, jnp.int32)])
                        p.append(ea_k * eb_k)
                    s123 = p[1] + p[2] + p[3]
                    gates_v[pl.ds(t * B + g * 16, 16)] = s123 / (p[0] + s123)

            rv = rows_b[b]
            gbase = t * B

            @plsc.parallel_loop(0, B, 1, unroll=16)
            def scale_row(e2):
                gsc = plsc.load_gather(
                    gates_v, [jnp.full((16,), 0, jnp.int32) + (gbase + e2)])
                for j in range(DH // 16):
                    rv[e2, pl.ds(j * 16, 16)] = rv[e2, pl.ds(j * 16, 16)] * gsc

            pltpu.make_async_copy(rows_b[b], agg_sh.at[dsts_v.at[t]],
                                  sems_b[b]).start(add=True)

        issue(0, 0)

        def pair(p, carry):
            t1 = 2 * p + 1
            issue(t1, 1)
            process(2 * p, 0)
            issue(t1 + 1, 0)
            process(t1, 1)
            return carry

        lax.fori_loop(0, (CH - 1) // 2, pair, 0)
        process(CH - 1, 0)
        pltpu.make_async_copy(rows_b[0], agg_sh.at[dsts_v.at[CH - 1]],
                              sems_b[0]).wait()
        pltpu.make_async_copy(rows_b[1], agg_sh.at[dsts_v.at[CH - 2]],
                              sems_b[1]).wait()
        plsc.subcore_barrier()
        pltpu.sync_copy(agg_sh.at[pl.ds(base_n, RPT)],
                        out_hbm.at[cid, half, pl.ds(base_n, RPT)])
        plsc.subcore_barrier()


@functools.cache
def _sc_agg():
    return pl.kernel(
        _sc_body,
        out_type=jax.ShapeDtypeStruct((NC, 2, N, DH), jnp.float32),
        mesh=plsc.VectorSubcoreMesh(core_axis_name="c", subcore_axis_name="s",
                                    num_cores=NC, num_subcores=NS),
        scratch_types=[
            pltpu.VMEM_SHARED((N, DH), jnp.float32),
            pltpu.VMEM((CH, B), jnp.int32),
            pltpu.VMEM((CH, B), jnp.int32),
            pltpu.VMEM((EPW,), jnp.float32),
            pltpu.VMEM((B, AB), jnp.float32),
            pltpu.VMEM((B, AB), jnp.float32),
            pltpu.VMEM((B, DH), jnp.float32),
            pltpu.VMEM((B, AB), jnp.float32),
            pltpu.VMEM((B, AB), jnp.float32),
            pltpu.VMEM((B, DH), jnp.float32),
            pltpu.SemaphoreType.DMA,
            pltpu.SemaphoreType.DMA,
            pltpu.SemaphoreType.DMA,
            pltpu.SemaphoreType.DMA,
            pltpu.SemaphoreType.DMA,
        ],
        compiler_params=pltpu.CompilerParams(needs_layout_passes=False,
                                             use_tc_tiling_on_sc=False),
    )


@jax.jit
def kernel(x, edge_index, W_rel, W_msg, W_self, W_upd, W_dec):
    wr2 = jnp.concatenate(
        [W_rel[:D], W_rel[D:], jnp.zeros((D, AB - 8), jnp.float32)], axis=1)
    wp = jnp.transpose(W_dec.reshape(D, K, T, F), (1, 3, 0, 2))

    y0, y1, h0, ab = pl.pallas_call(
        _pre_body,
        grid=(N // BR,),
        in_specs=[
            pl.BlockSpec((BR, D), lambda i: (i, 0)),
            pl.BlockSpec((D, D), lambda i: (0, 0)),
            pl.BlockSpec((D, D), lambda i: (0, 0)),
            pl.BlockSpec((D, AB), lambda i: (0, 0)),
        ],
        out_specs=[
            pl.BlockSpec((BR, DH), lambda i: (i, 0)),
            pl.BlockSpec((BR, DH), lambda i: (i, 0)),
            pl.BlockSpec((BR, D), lambda i: (i, 0)),
            pl.BlockSpec((BR, AB), lambda i: (i, 0)),
        ],
        out_shape=[
            jax.ShapeDtypeStruct((N, DH), jnp.float32),
            jax.ShapeDtypeStruct((N, DH), jnp.float32),
            jax.ShapeDtypeStruct((N, D), jnp.float32),
            jax.ShapeDtypeStruct((N, AB), jnp.float32),
        ],
    )(x, W_msg, W_self, wr2)

    parts = _sc_agg()(edge_index, y0, y1, ab)

    h = pl.pallas_call(
        _tail_body,
        grid=(N // BR,),
        in_specs=[
            pl.BlockSpec((BR, D), lambda i: (i, 0)),
            pl.BlockSpec((1, 1, BR, DH), lambda i: (0, 0, i, 0)),
            pl.BlockSpec((1, 1, BR, DH), lambda i: (0, 1, i, 0)),
            pl.BlockSpec((1, 1, BR, DH), lambda i: (1, 0, i, 0)),
            pl.BlockSpec((1, 1, BR, DH), lambda i: (1, 1, i, 0)),
            pl.BlockSpec((DH, D), lambda i: (0, 0)),
            pl.BlockSpec((DH, D), lambda i: (1, 0)),
        ],
        out_specs=pl.BlockSpec((BR, D), lambda i: (i, 0)),
        out_shape=jax.ShapeDtypeStruct((N, D), jnp.float32),
    )(h0, parts, parts, parts, parts, W_upd, W_upd)

    out4 = pl.pallas_call(
        _dec_body,
        grid=(K,),
        in_specs=[
            pl.BlockSpec((N, D), lambda k: (0, 0)),
            pl.BlockSpec((1, F, D, T), lambda k: (k, 0, 0, 0)),
        ],
        out_specs=pl.BlockSpec((1, F, T, N), lambda k: (k, 0, 0, 0)),
        out_shape=jax.ShapeDtypeStruct((K, F, T, N), jnp.float32),
    )(h, wp)

    return jnp.transpose(out4, (3, 0, 2, 1))


# speedup vs baseline: 1.4790x; 1.0067x over previous
import functools

import jax
import jax.numpy as jnp
from jax import lax
from jax.experimental import pallas as pl
from jax.experimental.pallas import tpu as pltpu
from jax.experimental.pallas import tpu_sc as plsc

N = 10000
E = 320000
D = 128
DH = 64
AB = 16
K = 6
T = 30
F = 5

NC = 2
NS = 16
NW = NC * NS
EPW = E // NW
B = 80
CH = EPW // B
RSTEP = 624
RPT = 640
BR = 1000


def _pre_body(x_ref, wmsg_ref, wself_ref, wr2_ref, y0_ref, y1_ref, h0_ref,
              ab_ref):
    xb = x_ref[...]
    ym = jnp.dot(xb, wmsg_ref[...], preferred_element_type=jnp.float32)
    y0_ref[...] = ym[:, :DH]
    y1_ref[...] = ym[:, DH:]
    h0_ref[...] = jnp.dot(xb, wself_ref[...], preferred_element_type=jnp.float32)
    ab_ref[...] = jnp.exp(
        jnp.dot(xb, wr2_ref[...], preferred_element_type=jnp.float32))


def _tail_body(h0_ref, p00_ref, p01_ref, p10_ref, p11_ref, wu0_ref, wu1_ref,
               h_ref):
    agg0 = p00_ref[0, 0] + p10_ref[0, 0]
    agg1 = p01_ref[0, 0] + p11_ref[0, 0]
    h = (h0_ref[...]
         + jnp.dot(agg0, wu0_ref[...], preferred_element_type=jnp.float32)
         + jnp.dot(agg1, wu1_ref[...], preferred_element_type=jnp.float32))
    h_ref[...] = jnp.maximum(h, 0.0)


def _dec_body(h_ref, wp_ref, out_ref):
    for f in range(F):
        out_ref[0, f] = lax.dot_general(
            wp_ref[0, f], h_ref[...], (((0,), (1,)), ((), ())),
            preferred_element_type=jnp.float32)


def _sc_body(ei_hbm, y0_hbm, y1_hbm, ab_hbm, out_hbm,
             agg_sh, srcs_v, dsts_v, gates_v, abs0, abd0, rows0,
             abs1, abd1, rows1,
             sem0, sem1, sems0, sems1, semi):
    cid = lax.axis_index("c")
    sid = lax.axis_index("s")
    wid = cid * NS + sid
    base_n = sid * RSTEP

    def _pre_issue(t, c):
        off = wid * EPW + t * B
        pltpu.async_copy(ei_hbm.at[0, pl.ds(off, B)], srcs_v.at[t], semi)
        pltpu.async_copy(ei_hbm.at[1, pl.ds(off, B)], dsts_v.at[t], semi)
        return c
    lax.fori_loop(0, CH, _pre_issue, 0)

    def _pre_drain(t, c):
        off = wid * EPW + t * B
        pltpu.make_async_copy(ei_hbm.at[0, pl.ds(off, B)], srcs_v.at[t],
                              semi).wait()
        pltpu.make_async_copy(ei_hbm.at[1, pl.ds(off, B)], dsts_v.at[t],
                              semi).wait()
        return c
    lax.fori_loop(0, CH, _pre_drain, 0)

    abs_b, abd_b = (abs0, abs1), (abd0, abd1)
    rows_b = (rows0, rows1)
    sem_b, sems_b = (sem0, sem1), (sems0, sems1)

    for half in range(2):
        yh_hbm = y0_hbm if half == 0 else y1_hbm
        def _zrow(r, c):
            for j in range(DH // 16):
                rows0[r, pl.ds(j * 16, 16)] = jnp.zeros((16,), jnp.float32)
            return c
        lax.fori_loop(0, B, _zrow, 0)
        for i in range(RPT // B):
            pltpu.sync_copy(rows0, agg_sh.at[pl.ds(base_n + i * B, B)])
        plsc.subcore_barrier()

        def issue(t, b):
            @pl.when(t >= 2)
            def _():
                pltpu.make_async_copy(rows_b[b], agg_sh.at[dsts_v.at[t]],
                                      sems_b[b]).wait()
            if half == 0:
                pltpu.async_copy(ab_hbm.at[srcs_v.at[t]], abs_b[b], sem_b[b])
                pltpu.async_copy(ab_hbm.at[dsts_v.at[t]], abd_b[b], sem_b[b])
            pltpu.async_copy(yh_hbm.at[srcs_v.at[t]], rows_b[b], sem_b[b])

        def process(t, b):
            if half == 0:
                pltpu.make_async_copy(ab_hbm.at[srcs_v.at[t]], abs_b[b],
                                      sem_b[b]).wait()
                pltpu.make_async_copy(ab_hbm.at[dsts_v.at[t]], abd_b[b],
                                      sem_b[b]).wait()
            pltpu.make_async_copy(yh_hbm.at[srcs_v.at[t]], rows_b[b],
                                  sem_b[b]).wait()
            if half == 0:
                for g in range(B // 16):
                    ev = jnp.arange(16, dtype=jnp.int32) + (g * 16)
                    p = []
                    for k in range(4):
                        ea_k = plsc.load_gather(
                            abs_b[b], [ev, jnp.full((16,), k, jnp.int32)])
                        eb_k = plsc.load_gather(
                            abd_b[b], [ev, jnp.full((16,), 4 + k, jnp.int32)])
                        p.append(ea_k * eb_k)
                    s123 = p[1] + p[2] + p[3]
                    gates_v[pl.ds(t * B + g * 16, 16)] = s123 / (p[0] + s123)

            rv = rows_b[b]
            gbase = t * B

            @plsc.parallel_loop(0, B, 1, unroll=16)
            def scale_row(e2):
                gsc = gates_v[pl.ds(gbase + e2, 16)][0]
                for j in range(DH // 16):
                    rv[e2, pl.ds(j * 16, 16)] = rv[e2, pl.ds(j * 16, 16)] * gsc

            pltpu.make_async_copy(rows_b[b], agg_sh.at[dsts_v.at[t]],
                                  sems_b[b]).start(add=True)

        issue(0, 0)

        def pair(p, carry):
            t1 = 2 * p + 1
            issue(t1, 1)
            process(2 * p, 0)
            issue(t1 + 1, 0)
            process(t1, 1)
            return carry

        lax.fori_loop(0, (CH - 1) // 2, pair, 0)
        process(CH - 1, 0)
        pltpu.make_async_copy(rows_b[0], agg_sh.at[dsts_v.at[CH - 1]],
                              sems_b[0]).wait()
        pltpu.make_async_copy(rows_b[1], agg_sh.at[dsts_v.at[CH - 2]],
                              sems_b[1]).wait()
        plsc.subcore_barrier()
        pltpu.sync_copy(agg_sh.at[pl.ds(base_n, RPT)],
                        out_hbm.at[cid, half, pl.ds(base_n, RPT)])
        plsc.subcore_barrier()


@functools.cache
def _sc_agg():
    return pl.kernel(
        _sc_body,
        out_type=jax.ShapeDtypeStruct((NC, 2, N, DH), jnp.float32),
        mesh=plsc.VectorSubcoreMesh(core_axis_name="c", subcore_axis_name="s",
                                    num_cores=NC, num_subcores=NS),
        scratch_types=[
            pltpu.VMEM_SHARED((N, DH), jnp.float32),
            pltpu.VMEM((CH, B), jnp.int32),
            pltpu.VMEM((CH, B), jnp.int32),
            pltpu.VMEM((EPW + 16,), jnp.float32),
            pltpu.VMEM((B, AB), jnp.float32),
            pltpu.VMEM((B, AB), jnp.float32),
            pltpu.VMEM((B, DH), jnp.float32),
            pltpu.VMEM((B, AB), jnp.float32),
            pltpu.VMEM((B, AB), jnp.float32),
            pltpu.VMEM((B, DH), jnp.float32),
            pltpu.SemaphoreType.DMA,
            pltpu.SemaphoreType.DMA,
            pltpu.SemaphoreType.DMA,
            pltpu.SemaphoreType.DMA,
            pltpu.SemaphoreType.DMA,
        ],
        compiler_params=pltpu.CompilerParams(needs_layout_passes=False,
                                             use_tc_tiling_on_sc=False),
    )


@jax.jit
def kernel(x, edge_index, W_rel, W_msg, W_self, W_upd, W_dec):
    wr2 = jnp.concatenate(
        [W_rel[:D], W_rel[D:], jnp.zeros((D, AB - 8), jnp.float32)], axis=1)
    wp = jnp.transpose(W_dec.reshape(D, K, T, F), (1, 3, 0, 2))

    y0, y1, h0, ab = pl.pallas_call(
        _pre_body,
        grid=(N // BR,),
        in_specs=[
            pl.BlockSpec((BR, D), lambda i: (i, 0)),
            pl.BlockSpec((D, D), lambda i: (0, 0)),
            pl.BlockSpec((D, D), lambda i: (0, 0)),
            pl.BlockSpec((D, AB), lambda i: (0, 0)),
        ],
        out_specs=[
            pl.BlockSpec((BR, DH), lambda i: (i, 0)),
            pl.BlockSpec((BR, DH), lambda i: (i, 0)),
            pl.BlockSpec((BR, D), lambda i: (i, 0)),
            pl.BlockSpec((BR, AB), lambda i: (i, 0)),
        ],
        out_shape=[
            jax.ShapeDtypeStruct((N, DH), jnp.float32),
            jax.ShapeDtypeStruct((N, DH), jnp.float32),
            jax.ShapeDtypeStruct((N, D), jnp.float32),
            jax.ShapeDtypeStruct((N, AB), jnp.float32),
        ],
    )(x, W_msg, W_self, wr2)

    parts = _sc_agg()(edge_index, y0, y1, ab)

    h = pl.pallas_call(
        _tail_body,
        grid=(N // BR,),
        in_specs=[
            pl.BlockSpec((BR, D), lambda i: (i, 0)),
            pl.BlockSpec((1, 1, BR, DH), lambda i: (0, 0, i, 0)),
            pl.BlockSpec((1, 1, BR, DH), lambda i: (0, 1, i, 0)),
            pl.BlockSpec((1, 1, BR, DH), lambda i: (1, 0, i, 0)),
            pl.BlockSpec((1, 1, BR, DH), lambda i: (1, 1, i, 0)),
            pl.BlockSpec((DH, D), lambda i: (0, 0)),
            pl.BlockSpec((DH, D), lambda i: (1, 0)),
        ],
        out_specs=pl.BlockSpec((BR, D), lambda i: (i, 0)),
        out_shape=jax.ShapeDtypeStruct((N, D), jnp.float32),
    )(h0, parts, parts, parts, parts, W_upd, W_upd)

    out4 = pl.pallas_call(
        _dec_body,
        grid=(K,),
        in_specs=[
            pl.BlockSpec((N, D), lambda k: (0, 0)),
            pl.BlockSpec((1, F, D, T), lambda k: (k, 0, 0, 0)),
        ],
        out_specs=pl.BlockSpec((1, F, T, N), lambda k: (k, 0, 0, 0)),
        out_shape=jax.ShapeDtypeStruct((K, F, T, N), jnp.float32),
    )(h, wp)

    return jnp.transpose(out4, (3, 0, 2, 1))


# speedup vs baseline: 1.5112x; 1.0217x over previous
import functools

import jax
import jax.numpy as jnp
from jax import lax
from jax.experimental import pallas as pl
from jax.experimental.pallas import tpu as pltpu
from jax.experimental.pallas import tpu_sc as plsc

N = 10000
E = 320000
D = 128
DH = 64
AB = 16
K = 6
T = 30
F = 5

NC = 2
NS = 16
NW = NC * NS
EPW = E // NW
B = 80
CH = EPW // B
RSTEP = 624
RPT = 640
BR = 1000


def _pre_body(x_ref, wmsg_ref, wself_ref, wr2_ref, y0_ref, y1_ref, h0_ref,
              ab_ref):
    xb = x_ref[...]
    ym = jnp.dot(xb, wmsg_ref[...], preferred_element_type=jnp.float32)
    y0_ref[...] = ym[:, :DH]
    y1_ref[...] = ym[:, DH:]
    h0_ref[...] = jnp.dot(xb, wself_ref[...], preferred_element_type=jnp.float32)
    ab_ref[...] = jnp.exp(
        jnp.dot(xb, wr2_ref[...], preferred_element_type=jnp.float32))


def _dec_body(h0_ref, p00_ref, p01_ref, p10_ref, p11_ref, wu0_ref, wu1_ref,
              wp_ref, out_ref, h_s):
    @pl.when(pl.program_id(0) == 0)
    def _():
        agg0 = p00_ref[0, 0] + p10_ref[0, 0]
        agg1 = p01_ref[0, 0] + p11_ref[0, 0]
        h = (h0_ref[...]
             + jnp.dot(agg0, wu0_ref[...], preferred_element_type=jnp.float32)
             + jnp.dot(agg1, wu1_ref[...], preferred_element_type=jnp.float32))
        h_s[...] = jnp.maximum(h, 0.0)
    for f in range(F):
        out_ref[0, f] = lax.dot_general(
            wp_ref[0, f], h_s[...], (((0,), (1,)), ((), ())),
            preferred_element_type=jnp.float32)


def _sc_body(ei_hbm, y0_hbm, y1_hbm, ab_hbm, out_hbm,
             agg_sh, srcs_v, dsts_v, gates_v, abs0, abd0, rows0,
             abs1, abd1, rows1,
             sem0, sem1, sems0, sems1, semi):
    cid = lax.axis_index("c")
    sid = lax.axis_index("s")
    wid = cid * NS + sid
    base_n = sid * RSTEP

    def _pre_issue(t, c):
        off = wid * EPW + t * B
        pltpu.async_copy(ei_hbm.at[0, pl.ds(off, B)], srcs_v.at[t], semi)
        pltpu.async_copy(ei_hbm.at[1, pl.ds(off, B)], dsts_v.at[t], semi)
        return c
    lax.fori_loop(0, CH, _pre_issue, 0)

    def _pre_drain(t, c):
        off = wid * EPW + t * B
        pltpu.make_async_copy(ei_hbm.at[0, pl.ds(off, B)], srcs_v.at[t],
                              semi).wait()
        pltpu.make_async_copy(ei_hbm.at[1, pl.ds(off, B)], dsts_v.at[t],
                              semi).wait()
        return c
    lax.fori_loop(0, CH, _pre_drain, 0)

    abs_b, abd_b = (abs0, abs1), (abd0, abd1)
    rows_b = (rows0, rows1)
    sem_b, sems_b = (sem0, sem1), (sems0, sems1)

    for half in range(2):
        yh_hbm = y0_hbm if half == 0 else y1_hbm
        def _zrow(r, c):
            for j in range(DH // 16):
                rows0[r, pl.ds(j * 16, 16)] = jnp.zeros((16,), jnp.float32)
            return c
        lax.fori_loop(0, B, _zrow, 0)
        for i in range(RPT // B):
            pltpu.sync_copy(rows0, agg_sh.at[pl.ds(base_n + i * B, B)])
        plsc.subcore_barrier()

        def issue(t, b):
            @pl.when(t >= 2)
            def _():
                pltpu.make_async_copy(rows_b[b], agg_sh.at[dsts_v.at[t]],
                                      sems_b[b]).wait()
            if half == 0:
                pltpu.async_copy(ab_hbm.at[srcs_v.at[t]], abs_b[b], sem_b[b])
                pltpu.async_copy(ab_hbm.at[dsts_v.at[t]], abd_b[b], sem_b[b])
            pltpu.async_copy(yh_hbm.at[srcs_v.at[t]], rows_b[b], sem_b[b])

        def process(t, b):
            if half == 0:
                pltpu.make_async_copy(ab_hbm.at[srcs_v.at[t]], abs_b[b],
                                      sem_b[b]).wait()
                pltpu.make_async_copy(ab_hbm.at[dsts_v.at[t]], abd_b[b],
                                      sem_b[b]).wait()
            pltpu.make_async_copy(yh_hbm.at[srcs_v.at[t]], rows_b[b],
                                  sem_b[b]).wait()
            if half == 0:
                for g in range(B // 16):
                    ev = jnp.arange(16, dtype=jnp.int32) + (g * 16)
                    p = []
                    for k in range(4):
                        ea_k = plsc.load_gather(
                            abs_b[b], [ev, jnp.full((16,), k, jnp.int32)])
                        eb_k = plsc.load_gather(
                            abd_b[b], [ev, jnp.full((16,), 4 + k, jnp.int32)])
                        p.append(ea_k * eb_k)
                    s123 = p[1] + p[2] + p[3]
                    gates_v[pl.ds(t * B + g * 16, 16)] = s123 / (p[0] + s123)

            rv = rows_b[b]
            gbase = t * B

            @plsc.parallel_loop(0, B, 1, unroll=16)
            def scale_row(e2):
                gsc = gates_v[pl.ds(gbase + e2, 16)][0]
                for j in range(DH // 16):
                    rv[e2, pl.ds(j * 16, 16)] = rv[e2, pl.ds(j * 16, 16)] * gsc

            pltpu.make_async_copy(rows_b[b], agg_sh.at[dsts_v.at[t]],
                                  sems_b[b]).start(add=True)

        issue(0, 0)

        def pair(p, carry):
            t1 = 2 * p + 1
            issue(t1, 1)
            process(2 * p, 0)
            issue(t1 + 1, 0)
            process(t1, 1)
            return carry

        lax.fori_loop(0, (CH - 1) // 2, pair, 0)
        process(CH - 1, 0)
        pltpu.make_async_copy(rows_b[0], agg_sh.at[dsts_v.at[CH - 1]],
                              sems_b[0]).wait()
        pltpu.make_async_copy(rows_b[1], agg_sh.at[dsts_v.at[CH - 2]],
                              sems_b[1]).wait()
        plsc.subcore_barrier()
        pltpu.sync_copy(agg_sh.at[pl.ds(base_n, RPT)],
                        out_hbm.at[cid, half, pl.ds(base_n, RPT)])
        plsc.subcore_barrier()


@functools.cache
def _sc_agg():
    return pl.kernel(
        _sc_body,
        out_type=jax.ShapeDtypeStruct((NC, 2, N, DH), jnp.float32),
        mesh=plsc.VectorSubcoreMesh(core_axis_name="c", subcore_axis_name="s",
                                    num_cores=NC, num_subcores=NS),
        scratch_types=[
            pltpu.VMEM_SHARED((N, DH), jnp.float32),
            pltpu.VMEM((CH, B), jnp.int32),
            pltpu.VMEM((CH, B), jnp.int32),
            pltpu.VMEM((EPW + 16,), jnp.float32),
            pltpu.VMEM((B, AB), jnp.float32),
            pltpu.VMEM((B, AB), jnp.float32),
            pltpu.VMEM((B, DH), jnp.float32),
            pltpu.VMEM((B, AB), jnp.float32),
            pltpu.VMEM((B, AB), jnp.float32),
            pltpu.VMEM((B, DH), jnp.float32),
            pltpu.SemaphoreType.DMA,
            pltpu.SemaphoreType.DMA,
            pltpu.SemaphoreType.DMA,
            pltpu.SemaphoreType.DMA,
            pltpu.SemaphoreType.DMA,
        ],
        compiler_params=pltpu.CompilerParams(needs_layout_passes=False,
                                             use_tc_tiling_on_sc=False),
    )


@jax.jit
def kernel(x, edge_index, W_rel, W_msg, W_self, W_upd, W_dec):
    wr2 = jnp.concatenate(
        [W_rel[:D], W_rel[D:], jnp.zeros((D, AB - 8), jnp.float32)], axis=1)
    wp = jnp.transpose(W_dec.reshape(D, K, T, F), (1, 3, 0, 2))

    y0, y1, h0, ab = pl.pallas_call(
        _pre_body,
        grid=(N // BR,),
        in_specs=[
            pl.BlockSpec((BR, D), lambda i: (i, 0)),
            pl.BlockSpec((D, D), lambda i: (0, 0)),
            pl.BlockSpec((D, D), lambda i: (0, 0)),
            pl.BlockSpec((D, AB), lambda i: (0, 0)),
        ],
        out_specs=[
            pl.BlockSpec((BR, DH), lambda i: (i, 0)),
            pl.BlockSpec((BR, DH), lambda i: (i, 0)),
            pl.BlockSpec((BR, D), lambda i: (i, 0)),
            pl.BlockSpec((BR, AB), lambda i: (i, 0)),
        ],
        out_shape=[
            jax.ShapeDtypeStruct((N, DH), jnp.float32),
            jax.ShapeDtypeStruct((N, DH), jnp.float32),
            jax.ShapeDtypeStruct((N, D), jnp.float32),
            jax.ShapeDtypeStruct((N, AB), jnp.float32),
        ],
    )(x, W_msg, W_self, wr2)

    parts = _sc_agg()(edge_index, y0, y1, ab)

    out4 = pl.pallas_call(
        _dec_body,
        grid=(K,),
        in_specs=[
            pl.BlockSpec((N, D), lambda k: (0, 0)),
            pl.BlockSpec((1, 1, N, DH), lambda k: (0, 0, 0, 0)),
            pl.BlockSpec((1, 1, N, DH), lambda k: (0, 1, 0, 0)),
            pl.BlockSpec((1, 1, N, DH), lambda k: (1, 0, 0, 0)),
            pl.BlockSpec((1, 1, N, DH), lambda k: (1, 1, 0, 0)),
            pl.BlockSpec((DH, D), lambda k: (0, 0)),
            pl.BlockSpec((DH, D), lambda k: (1, 0)),
            pl.BlockSpec((1, F, D, T), lambda k: (k, 0, 0, 0)),
        ],
        out_specs=pl.BlockSpec((1, F, T, N), lambda k: (k, 0, 0, 0)),
        out_shape=jax.ShapeDtypeStruct((K, F, T, N), jnp.float32),
        scratch_shapes=[pltpu.VMEM((N, D), jnp.float32)],
    )(h0, parts, parts, parts, parts, W_upd, W_upd, wp)

    return jnp.transpose(out4, (3, 0, 2, 1))
